# baseline (speedup 1.0000x reference)
import functools

import jax
import jax.numpy as jnp
from jax.experimental import pallas as pl
from jax.experimental.pallas import tpu as pltpu
from jax.experimental.pallas import tpu_sc as plsc

N_E = 11965
N_C = 188
D = 128
B = 128
L = 400
LB = B * L

C_PAD = 256
NP = 12288
NODE_CHUNK = 2048
N_SLICE = 4
L_S = L // N_SLICE
ROWS_S = L_S * B
SEQ_CHUNK = 25
GW = 128

def _dot(a, b):
    return jax.lax.dot_general(a.astype(jnp.bfloat16), b.astype(jnp.bfloat16),
                               (((1,), (0,)), ((), ())),
                               preferred_element_type=jnp.float32)


def _dot_t(a, b):
    return jax.lax.dot_general(a.astype(jnp.bfloat16), b.astype(jnp.bfloat16),
                               (((0,), (0,)), ((), ())),
                               preferred_element_type=jnp.float32)


_IDX_OFFS = (0, 100, 200, 0, 2, 9, 19, 0)


def _hg_body(he_ref, e_ref, whg_ref, i0, i1, i2, i3, i4, i5, i6, i7,
             out_ref, idxt_ref, m_scr, deg_scr):
    for k, ref in enumerate((i0, i1, i2, i3, i4, i5, i6, i7)):
        idxt_ref[k] = jnp.swapaxes(ref[...], 0, 1) + _IDX_OFFS[k]
    iota_c = jax.lax.broadcasted_iota(jnp.int32, (C_PAD, 1), 0)
    m_scr[...] = jnp.zeros_like(m_scr)
    deg_scr[...] = jnp.zeros_like(deg_scr)
    ones_col = jnp.ones((2 * NODE_CHUNK, 1), jnp.float32)

    def p_blk(i):
        sl = pl.ds(i * 2 * NODE_CHUNK, 2 * NODE_CHUNK)
        return (he_ref[:, sl] == iota_c).astype(jnp.bfloat16)

    def acc_body(i, carry):
        sl = pl.ds(i * NODE_CHUNK, NODE_CHUNK)
        e_blk = e_ref[sl, :]
        e2 = jnp.broadcast_to(e_blk[:, None, :], (NODE_CHUNK, 2, D))
        p = p_blk(i)
        m_scr[...] += _dot(p, e2.reshape(2 * NODE_CHUNK, D))
        deg_scr[...] += _dot(p, ones_col)
        return carry

    jax.lax.fori_loop(0, NP // NODE_CHUNK, acc_body, 0)
    m_scr[...] = m_scr[...] / jnp.maximum(deg_scr[...], 1.0)

    def out_body(i, carry):
        sl = pl.ds(i * NODE_CHUNK, NODE_CHUNK)
        pe = _dot_t(p_blk(i), m_scr[...])
        pe3 = pe.reshape(NODE_CHUNK, 2, D)
        agg = (pe3[:, 0, :] + pe3[:, 1, :]) * 0.5
        out_ref[sl, :] = jax.nn.relu(_dot(agg, whg_ref[...])) + e_ref[sl, :]
        return carry

    jax.lax.fori_loop(0, NP // NODE_CHUNK, out_body, 0)


def _hg_conv(he_row, e_pad, w_hg, idx_arrs):
    return pl.pallas_call(
        _hg_body,
        out_shape=[jax.ShapeDtypeStruct((NP, D), jnp.float32),
                   jax.ShapeDtypeStruct((8, L, B), jnp.int32)],
        scratch_shapes=[pltpu.VMEM((C_PAD, D), jnp.float32),
                        pltpu.VMEM((C_PAD, 1), jnp.float32)],
    )(he_row, e_pad, w_hg, *idx_arrs)


def _sc_gather(table, idx2d):
    n_idx = idx2d.shape[1]
    width = table.shape[1]
    mesh = plsc.VectorSubcoreMesh(core_axis_name="c", subcore_axis_name="s")

    @functools.partial(
        pl.kernel,
        out_type=jax.ShapeDtypeStruct((n_idx, width), table.dtype),
        mesh=mesh)
    def _gather_kernel(x_hbm, i_hbm, o_hbm):
        def body(i_vmem, o_vmem):
            pltpu.sync_copy(x_hbm.at[i_vmem.at[0]], o_vmem)

        pltpu.emit_pipeline(
            body,
            grid=(n_idx // GW,),
            in_specs=[pl.BlockSpec((1, GW), index_map=lambda i: (0, i))],
            out_specs=[pl.BlockSpec((GW, width), index_map=lambda i: (i, 0))],
            core_axis_name=("c", "s"),
            dimension_semantics=(pltpu.PARALLEL,),
        )(i_hbm, o_hbm)

    return _gather_kernel(table, idx2d)


MLP_ROWS = SEQ_CHUNK * B


_RSQRT_D = 1.0 / (128.0 ** 0.5)


def _sig(v):
    return 0.5 * jnp.tanh(0.5 * v) + 0.5


def _slice_body(xg_ref, idx_ref, tx_ref, ta_ref, winx_ref, wina_ref, wx_ref,
                b_ref, wh_ref, wo_ref, hin_ref,
                ps_ref, pm_ref, hout_ref,
                h_scr, hist_scr, xwx_scr, x_scr):
    @pl.when(pl.program_id(0) == 0)
    def _():
        h_scr[...] = hin_ref[...]

    iota_x = jax.lax.broadcasted_iota(jnp.int32, (256, 1), 0)
    iota_a = jax.lax.broadcasted_iota(jnp.int32, (32, 1), 0)
    bf = jnp.bfloat16

    def row(k):
        return idx_ref[k:k + 1, :]

    ohx_t = ((row(0) == iota_x).astype(bf)
             + (row(1) == iota_x).astype(bf)
             + (row(2) == iota_x).astype(bf))
    oha_t = ((row(3) == iota_a).astype(bf)
             + (row(4) == iota_a).astype(bf)
             + (row(5) == iota_a).astype(bf)
             + (row(6) == iota_a).astype(bf))
    x = xg_ref[...] + _dot_t(ohx_t, tx_ref[...])
    a_emb = _dot_t(oha_t, ta_ref[...])
    inter = jnp.tanh(_dot(x, winx_ref[...]) + _dot(a_emb, wina_ref[...]))
    xwx = _dot(inter, wx_ref[...]) + b_ref[...]
    x_scr[...] = x.reshape(SEQ_CHUNK, B, D)
    xwx_scr[...] = xwx.reshape(SEQ_CHUNK, B, 3 * D)

    ones_col = jnp.full((D, 1), _RSQRT_D, jnp.float32)
    wh_zr = wh_ref[:, :2 * D]
    wh_g = wh_ref[:, 2 * D:]
    wo = wo_ref[...]
    h = h_scr[...]
    for t in range(SEQ_CHUNK):
        xwx_t = xwx_scr[t]
        hb = h.astype(bf)
        zr = _sig(xwx_t[:, :2 * D] + _dot(hb, wh_zr))
        c = xwx_t[:, 2 * D:] + _dot(hb, wh_g)
        z = zr[:, :D]
        g = jnp.tanh(zr[:, D:] * c)
        h = h + z * (g - h)
        hist_scr[t] = h
    h_scr[...] = h
    hout_ref[...] = h
    for t in range(SEQ_CHUNK):
        ht = hist_scr[t]
        pm_ref[0, :, t:t + 1] = _dot(ht * x_scr[t], ones_col)
        ps_ref[0, :, t:t + 1] = _dot(ht, wo)
    ps_ref[0] = _sig(ps_ref[0])
    pm_ref[0] = _sig(pm_ref[0])


def _slice_kernel(xg, idx7, s, tx, ta, winx, wina, wx, b2d, wh, wo_col, h_in):
    n_chunks = L_S // SEQ_CHUNK
    out_spec = pl.BlockSpec((1, B, SEQ_CHUNK), lambda i: (i, 0, 0))
    idx_spec = pl.BlockSpec((8, MLP_ROWS), lambda i: (0, i + s * n_chunks))

    def w_spec(shape):
        return pl.BlockSpec(shape, lambda i: (0, 0))

    return pl.pallas_call(
        _slice_body,
        grid=(n_chunks,),
        in_specs=[pl.BlockSpec((MLP_ROWS, D), lambda i: (i, 0)), idx_spec,
                  w_spec((256, D)), w_spec((32, D)), w_spec((D, D)),
                  w_spec((D, D)), w_spec((D, 3 * D)), w_spec((1, 3 * D)),
                  w_spec((D, 3 * D)), w_spec((D, 1)),
                  pl.BlockSpec((B, D), lambda i: (0, 0))],
        out_specs=[out_spec, out_spec, pl.BlockSpec((B, D), lambda i: (0, 0))],
        out_shape=[jax.ShapeDtypeStruct((n_chunks, B, SEQ_CHUNK), jnp.float32),
                   jax.ShapeDtypeStruct((n_chunks, B, SEQ_CHUNK), jnp.float32),
                   jax.ShapeDtypeStruct((B, D), jnp.float32)],
        scratch_shapes=[pltpu.VMEM((B, D), jnp.float32),
                        pltpu.VMEM((SEQ_CHUNK, B, D), jnp.float32),
                        pltpu.VMEM((SEQ_CHUNK, B, 3 * D), jnp.float32),
                        pltpu.VMEM((SEQ_CHUNK, B, D), jnp.float32)],
        compiler_params=pltpu.CompilerParams(
            dimension_semantics=("arbitrary",)),
    )(xg, idx7, tx, ta, winx, wina, wx, b2d, wh, wo_col, h_in)


def kernel(input_e, input_ed, input_ep, input_a, input_as, input_ha, input_ca,
           input_it, node_ids, he_ids,
           E_table, ED_table, EP_table, A_table, AS_table, HA_table, CA_table,
           IT_table, W_hg, W_in, Wx, Wh, b, w_out_s):
    f32 = jnp.float32
    he_row = jnp.pad(he_ids.astype(jnp.int32), (0, 2 * (NP - N_E)),
                     constant_values=200).reshape(1, 2 * NP)
    e_pad = jnp.zeros((NP, D), f32).at[:N_E].set(E_table.astype(f32))

    idx_arrs = [a.astype(jnp.int32)
                for a in (input_ed, input_ep, input_it, input_a, input_as,
                          input_ha, input_ca, input_e)]

    e_hg, idx8_t = _hg_conv(he_row, e_pad, W_hg.astype(f32), idx_arrs)
    idx8 = idx8_t.reshape(8, LB)
    idx_e = idx8[7:8]

    t_x = jnp.zeros((256, D), f32)
    t_x = t_x.at[0:100].set(ED_table.astype(f32))
    t_x = t_x.at[100:200].set(EP_table.astype(f32))
    t_x = t_x.at[200:207].set(IT_table.astype(f32))
    t_a = jnp.zeros((32, D), f32)
    t_a = t_a.at[0:2].set(A_table.astype(f32))
    t_a = t_a.at[2:9].set(AS_table.astype(f32))
    t_a = t_a.at[9:19].set(HA_table.astype(f32))
    t_a = t_a.at[19:29].set(CA_table.astype(f32))

    winx, wina = W_in[:D].astype(f32), W_in[D:].astype(f32)
    wx_f = Wx.astype(f32)
    b2d = b.reshape(1, 3 * D).astype(f32)
    wh_f = Wh.astype(f32)
    wo_col = w_out_s.reshape(D, 1).astype(f32)

    h = jnp.zeros((B, D), f32)
    ps_parts, pm_parts = [], []
    xgs = [_sc_gather(e_hg, idx_e[:, s * ROWS_S:(s + 1) * ROWS_S])
           for s in range(N_SLICE)]
    for s in range(N_SLICE):
        ps3, pm3, h = _slice_kernel(xgs[s], idx8, s, t_x, t_a, winx, wina,
                                    wx_f, b2d, wh_f, wo_col, h)
        ps_parts.append(ps3)
        pm_parts.append(pm3)
    ps_all = jnp.concatenate(ps_parts, axis=0)
    pm_all = jnp.concatenate(pm_parts, axis=0)
    pred_s = jnp.swapaxes(ps_all, 0, 1).reshape(B, L)
    pred_main = jnp.swapaxes(pm_all, 0, 1).reshape(B, L)
    return (pred_s, pred_main)


# speedup vs baseline: 1.0949x; 1.0949x over previous
import functools

import jax
import jax.numpy as jnp
from jax.experimental import pallas as pl
from jax.experimental.pallas import tpu as pltpu
from jax.experimental.pallas import tpu_sc as plsc

N_E = 11965
N_C = 188
D = 128
B = 128
L = 400
LB = B * L

C_PAD = 256
NP = 12288
NODE_CHUNK = 2048
N_SLICE = 4
L_S = L // N_SLICE
ROWS_S = L_S * B
SEQ_CHUNK = 20
MLP_ROWS = SEQ_CHUNK * B
GW = 128


def _dot(a, b):
    return jax.lax.dot_general(a.astype(jnp.bfloat16), b.astype(jnp.bfloat16),
                               (((1,), (0,)), ((), ())),
                               preferred_element_type=jnp.float32)


def _dot_t(a, b):
    return jax.lax.dot_general(a.astype(jnp.bfloat16), b.astype(jnp.bfloat16),
                               (((0,), (0,)), ((), ())),
                               preferred_element_type=jnp.float32)


def _hg_body(het_ref, e_ref, whg_ref, out_ref, m_scr, deg_scr):
    iota_c = jax.lax.broadcasted_iota(jnp.int32, (C_PAD, 1), 0)
    m_scr[...] = jnp.zeros_like(m_scr)
    deg_scr[...] = jnp.zeros_like(deg_scr)
    ones_col = jnp.ones((NODE_CHUNK, 1), jnp.float32)

    def ht_blk(i):
        sl = pl.ds(i * NODE_CHUNK, NODE_CHUNK)
        return ((het_ref[0:1, sl] == iota_c).astype(jnp.bfloat16)
                + (het_ref[1:2, sl] == iota_c).astype(jnp.bfloat16))

    def acc_body(i, carry):
        sl = pl.ds(i * NODE_CHUNK, NODE_CHUNK)
        ht = ht_blk(i)
        m_scr[...] += _dot(ht, e_ref[sl, :])
        deg_scr[...] += _dot(ht, ones_col)
        return carry

    jax.lax.fori_loop(0, NP // NODE_CHUNK, acc_body, 0)
    m_scr[...] = m_scr[...] / jnp.maximum(deg_scr[...], 1.0)

    def out_body(i, carry):
        sl = pl.ds(i * NODE_CHUNK, NODE_CHUNK)
        agg = _dot_t(ht_blk(i), m_scr[...]) * 0.5
        out_ref[sl, :] = jax.nn.relu(_dot(agg, whg_ref[...])) + e_ref[sl, :]
        return carry

    jax.lax.fori_loop(0, NP // NODE_CHUNK, out_body, 0)


def _hg_conv(het, e_pad, w_hg):
    return pl.pallas_call(
        _hg_body,
        out_shape=jax.ShapeDtypeStruct((NP, D), jnp.float32),
        scratch_shapes=[pltpu.VMEM((C_PAD, D), jnp.float32),
                        pltpu.VMEM((C_PAD, 1), jnp.float32)],
    )(het, e_pad, w_hg)


def _sc_gather(table, idx2d):
    n_idx = idx2d.shape[1]
    width = table.shape[1]
    mesh = plsc.VectorSubcoreMesh(core_axis_name="c", subcore_axis_name="s")

    @functools.partial(
        pl.kernel,
        out_type=jax.ShapeDtypeStruct((n_idx, width), table.dtype),
        mesh=mesh)
    def _gather_kernel(x_hbm, i_hbm, o_hbm):
        def body(i_vmem, o_vmem):
            pltpu.sync_copy(x_hbm.at[i_vmem.at[0]], o_vmem)

        pltpu.emit_pipeline(
            body,
            grid=(n_idx // GW,),
            in_specs=[pl.BlockSpec((1, GW), index_map=lambda i: (0, i))],
            out_specs=[pl.BlockSpec((GW, width), index_map=lambda i: (i, 0))],
            core_axis_name=("c", "s"),
            dimension_semantics=(pltpu.PARALLEL,),
        )(i_hbm, o_hbm)

    return _gather_kernel(table, idx2d)


_RSQRT_D = 1.0 / (128.0 ** 0.5)


def _sig(v):
    return 0.5 * jnp.tanh(0.5 * v) + 0.5


def _slice_body(xg_ref, idx_ref, tx_ref, ta_ref, winx_ref, wina_ref, wx_ref,
                b_ref, wh_ref, wo_ref, hin_ref,
                ps_ref, pm_ref, hout_ref,
                h_scr, hist_scr, xwx_scr, x_scr):
    @pl.when(pl.program_id(0) == 0)
    def _():
        h_scr[...] = hin_ref[...]

    iota_x = jax.lax.broadcasted_iota(jnp.int32, (256, 1), 0)
    iota_a = jax.lax.broadcasted_iota(jnp.int32, (32, 1), 0)
    bf = jnp.bfloat16

    def row(k):
        return idx_ref[k:k + 1, :]

    ohx_t = ((row(0) == iota_x).astype(bf)
             + (row(1) == iota_x).astype(bf)
             + (row(2) == iota_x).astype(bf))
    oha_t = ((row(3) == iota_a).astype(bf)
             + (row(4) == iota_a).astype(bf)
             + (row(5) == iota_a).astype(bf)
             + (row(6) == iota_a).astype(bf))
    x = xg_ref[...] + _dot_t(ohx_t, tx_ref[...])
    a_emb = _dot_t(oha_t, ta_ref[...])
    inter = jnp.tanh(_dot(x, winx_ref[...]) + _dot(a_emb, wina_ref[...]))
    xwx = _dot(inter, wx_ref[...]) + b_ref[...]
    x_scr[...] = x.reshape(SEQ_CHUNK, B, D)
    xwx_scr[...] = xwx.reshape(SEQ_CHUNK, B, 3 * D)

    ones_col = jnp.full((D, 1), _RSQRT_D, jnp.float32)
    wh_zr = wh_ref[:, :2 * D]
    wh_g = wh_ref[:, 2 * D:]
    wo = wo_ref[...]
    h = h_scr[...]
    for t in range(SEQ_CHUNK):
        xwx_t = xwx_scr[t]
        hb = h.astype(bf)
        zr = _sig(xwx_t[:, :2 * D] + _dot(hb, wh_zr))
        c = xwx_t[:, 2 * D:] + _dot(hb, wh_g)
        z = zr[:, :D]
        g = jnp.tanh(zr[:, D:] * c)
        h = h + z * (g - h)
        hist_scr[t] = h
    h_scr[...] = h
    hout_ref[...] = h
    for t in range(SEQ_CHUNK):
        ht = hist_scr[t]
        pm_ref[0, :, t:t + 1] = _dot(ht * x_scr[t], ones_col)
        ps_ref[0, :, t:t + 1] = _dot(ht, wo)
    ps_ref[0] = _sig(ps_ref[0])
    pm_ref[0] = _sig(pm_ref[0])


def _slice_kernel(xg, idx7, s, tx, ta, winx, wina, wx, b2d, wh, wo_col, h_in):
    n_chunks = L_S // SEQ_CHUNK
    out_spec = pl.BlockSpec((1, B, SEQ_CHUNK), lambda i: (i, 0, 0))
    idx_spec = pl.BlockSpec((7, MLP_ROWS), lambda i: (0, i + s * n_chunks))

    def w_spec(shape):
        return pl.BlockSpec(shape, lambda i: (0, 0))

    return pl.pallas_call(
        _slice_body,
        grid=(n_chunks,),
        in_specs=[pl.BlockSpec((MLP_ROWS, D), lambda i: (i, 0)), idx_spec,
                  w_spec((256, D)), w_spec((32, D)), w_spec((D, D)),
                  w_spec((D, D)), w_spec((D, 3 * D)), w_spec((1, 3 * D)),
                  w_spec((D, 3 * D)), w_spec((D, 1)),
                  pl.BlockSpec((B, D), lambda i: (0, 0))],
        out_specs=[out_spec, out_spec, pl.BlockSpec((B, D), lambda i: (0, 0))],
        out_shape=[jax.ShapeDtypeStruct((n_chunks, B, SEQ_CHUNK), jnp.float32),
                   jax.ShapeDtypeStruct((n_chunks, B, SEQ_CHUNK), jnp.float32),
                   jax.ShapeDtypeStruct((B, D), jnp.float32)],
        scratch_shapes=[pltpu.VMEM((B, D), jnp.float32),
                        pltpu.VMEM((SEQ_CHUNK, B, D), jnp.float32),
                        pltpu.VMEM((SEQ_CHUNK, B, 3 * D), jnp.float32),
                        pltpu.VMEM((SEQ_CHUNK, B, D), jnp.float32)],
        compiler_params=pltpu.CompilerParams(
            dimension_semantics=("arbitrary",)),
    )(xg, idx7, tx, ta, winx, wina, wx, b2d, wh, wo_col, h_in)


def kernel(input_e, input_ed, input_ep, input_a, input_as, input_ha, input_ca,
           input_it, node_ids, he_ids,
           E_table, ED_table, EP_table, A_table, AS_table, HA_table, CA_table,
           IT_table, W_hg, W_in, Wx, Wh, b, w_out_s):
    f32 = jnp.float32
    he2 = jnp.pad(he_ids.reshape(N_E, 2).astype(jnp.int32),
                  ((0, NP - N_E), (0, 0)), constant_values=200)
    het = jnp.swapaxes(he2, 0, 1)
    e_pad = jnp.zeros((NP, D), f32).at[:N_E].set(E_table.astype(f32))

    e_hg = _hg_conv(het, e_pad, W_hg.astype(f32))

    idx_e = jnp.swapaxes(input_e, 0, 1).reshape(N_SLICE, 1, ROWS_S).astype(
        jnp.int32)

    offs = jnp.array([0, 100, 200, 0, 2, 9, 19], jnp.int32)
    idx7 = (jnp.stack([input_ed, input_ep, input_it, input_a, input_as,
                       input_ha, input_ca]).astype(jnp.int32)
            + offs[:, None, None])
    idx7 = jnp.swapaxes(idx7, 1, 2).reshape(7, LB)

    t_x = jnp.zeros((256, D), f32)
    t_x = t_x.at[0:100].set(ED_table.astype(f32))
    t_x = t_x.at[100:200].set(EP_table.astype(f32))
    t_x = t_x.at[200:207].set(IT_table.astype(f32))
    t_a = jnp.zeros((32, D), f32)
    t_a = t_a.at[0:2].set(A_table.astype(f32))
    t_a = t_a.at[2:9].set(AS_table.astype(f32))
    t_a = t_a.at[9:19].set(HA_table.astype(f32))
    t_a = t_a.at[19:29].set(CA_table.astype(f32))

    winx, wina = W_in[:D].astype(f32), W_in[D:].astype(f32)
    wx_f = Wx.astype(f32)
    b2d = b.reshape(1, 3 * D).astype(f32)
    wh_f = Wh.astype(f32)
    wo_col = w_out_s.reshape(D, 1).astype(f32)

    h = jnp.zeros((B, D), f32)
    ps_parts, pm_parts = [], []
    xgs = [_sc_gather(e_hg, idx_e[s]) for s in range(N_SLICE)]
    for s in range(N_SLICE):
        ps3, pm3, h = _slice_kernel(xgs[s], idx7, s, t_x, t_a, winx, wina,
                                    wx_f, b2d, wh_f, wo_col, h)
        ps_parts.append(ps3)
        pm_parts.append(pm3)
    ps_all = jnp.concatenate(ps_parts, axis=0)
    pm_all = jnp.concatenate(pm_parts, axis=0)
    pred_s = jnp.swapaxes(ps_all, 0, 1).reshape(B, L)
    pred_main = jnp.swapaxes(pm_all, 0, 1).reshape(B, L)
    return (pred_s, pred_main)
